# baseline (device time: 31417 ns/iter reference)
import jax
import jax.numpy as jnp
from jax import lax
from jax.experimental import pallas as pl
from jax.experimental.pallas import tpu as pltpu

NR = 8
NT = 2 * NR
K_SHARD = 768
S_X = 5.0 * (K_SHARD ** 0.5) / 127.0
S_Y = 5.0 * ((2 * K_SHARD) ** 0.5) / 127.0


def _quant(v, scale):
    q = jnp.clip(jnp.round(v * (1.0 / scale)), -127.0, 127.0)
    return q.astype(jnp.int8)


def kernel(A, B):
    M, K = A.shape
    _, N = B.shape
    HALF = M // 2
    CH = HALF // NR
    CW = N // 2

    def body(a_ref, b_ref, out_ref,
             x_send, x_recv, y_send, y_recv, out_stage,
             x_send_sems, x_recv_sems, y_send_sems, y_recv_sems, copy_sems):
        my_x = lax.axis_index("x")
        my_y = lax.axis_index("y")
        x_peer = (1 - my_x, my_y)
        y_peer = (my_x, 1 - my_y)

        barrier_sem = pltpu.get_barrier_semaphore()
        for peer in (x_peer, y_peer):
            pl.semaphore_signal(
                barrier_sem, inc=1, device_id=peer,
                device_id_type=pl.DeviceIdType.MESH,
            )

        def x_rdma(t):
            return pltpu.make_async_remote_copy(
                src_ref=x_send.at[t], dst_ref=x_recv.at[t],
                send_sem=x_send_sems.at[t], recv_sem=x_recv_sems.at[t],
                device_id=x_peer, device_id_type=pl.DeviceIdType.MESH,
            )

        def y_rdma(t):
            return pltpu.make_async_remote_copy(
                src_ref=y_send.at[t], dst_ref=y_recv.at[t],
                send_sem=y_send_sems.at[t], recv_sem=y_recv_sems.at[t],
                device_id=y_peer, device_id_type=pl.DeviceIdType.MESH,
            )

        def out_copy(t, half_owner, sem):
            c, h = divmod(t, 2)
            rows = pl.ds(half_owner * HALF + c * CH, CH)
            return pltpu.make_async_copy(
                out_stage.at[sem],
                out_ref.at[rows, pl.ds(h * CW, CW)],
                copy_sems.at[sem],
            )

        b_halves = [None, None]

        partials = []
        for t in range(NT):
            c, h = divmod(t, 2)
            if b_halves[h] is None:
                b_halves[h] = b_ref[:, h * CW:(h + 1) * CW].astype(jnp.bfloat16)
            rows = pl.ds(my_y * HALF + c * CH, CH)
            p = jnp.dot(
                a_ref[rows, :].astype(jnp.bfloat16), b_halves[h],
                preferred_element_type=jnp.float32,
            )
            partials.append(p)
            x_send[t] = _quant(p, S_X)
            if t == 0:
                pl.semaphore_wait(barrier_sem, 2)
            x_rdma(t).start()

        for t in range(NT):
            x_rdma(t).wait_recv()
            red = partials[t] + x_recv[t].astype(jnp.float32) * S_X
            y_send[t] = _quant(red, S_Y)
            y_rdma(t).start()
            out_stage[t] = red.astype(jnp.bfloat16)
            out_copy(t, my_y, t).start()

        for t in range(NT):
            y_rdma(t).wait_recv()
            out_stage[NT + t] = (
                y_recv[t].astype(jnp.float32) * S_Y
            ).astype(jnp.bfloat16)
            out_copy(t, 1 - my_y, NT + t).start()

        for t in range(NT):
            out_copy(t, my_y, t).wait()
        for t in range(NT):
            out_copy(t, 1 - my_y, NT + t).wait()
        for t in range(NT):
            x_rdma(t).wait_send()
            y_rdma(t).wait_send()

    return pl.pallas_call(
        body,
        out_shape=jax.ShapeDtypeStruct((M, N), jnp.bfloat16),
        in_specs=[
            pl.BlockSpec(memory_space=pltpu.VMEM),
            pl.BlockSpec(memory_space=pltpu.VMEM),
        ],
        out_specs=pl.BlockSpec(memory_space=pltpu.MemorySpace.HBM),
        scratch_shapes=[
            pltpu.VMEM((NT, CH, CW), jnp.int8),
            pltpu.VMEM((NT, CH, CW), jnp.int8),
            pltpu.VMEM((NT, CH, CW), jnp.int8),
            pltpu.VMEM((NT, CH, CW), jnp.int8),
            pltpu.VMEM((2 * NT, CH, CW), jnp.bfloat16),
            pltpu.SemaphoreType.DMA((NT,)),
            pltpu.SemaphoreType.DMA((NT,)),
            pltpu.SemaphoreType.DMA((NT,)),
            pltpu.SemaphoreType.DMA((NT,)),
            pltpu.SemaphoreType.DMA((2 * NT,)),
        ],
        compiler_params=pltpu.CompilerParams(collective_id=0),
    )(A, B)


# device time: 28853 ns/iter; 1.0889x vs baseline; 1.0889x over previous
import jax
import jax.numpy as jnp
from jax import lax
from jax.experimental import pallas as pl
from jax.experimental.pallas import tpu as pltpu

NC = 6
K_GLOBAL_HALF = 768
S_X = 5.0 * (K_GLOBAL_HALF ** 0.5) / 127.0
S_Y = 5.0 * ((2 * K_GLOBAL_HALF) ** 0.5) / 127.0


def _quant(v, scale):
    q = jnp.clip(jnp.round(v * (1.0 / scale)), -127.0, 127.0)
    return q.astype(jnp.int8)


def kernel(A, B):
    M, K = A.shape
    _, N = B.shape
    HALF = M // 2
    CH = HALF // NC

    def body(a_ref, b_ref, out_ref,
             x_send, x_recv, y_send, y_recv, out_stage,
             x_send_sems, x_recv_sems, y_send_sems, y_recv_sems, copy_sems):
        my_x = lax.axis_index("x")
        my_y = lax.axis_index("y")
        x_peer = (1 - my_x, my_y)
        y_peer = (my_x, 1 - my_y)

        barrier_sem = pltpu.get_barrier_semaphore()
        for peer in (x_peer, y_peer):
            pl.semaphore_signal(
                barrier_sem, inc=1, device_id=peer,
                device_id_type=pl.DeviceIdType.MESH,
            )

        def x_rdma(c):
            return pltpu.make_async_remote_copy(
                src_ref=x_send.at[c], dst_ref=x_recv.at[c],
                send_sem=x_send_sems.at[c], recv_sem=x_recv_sems.at[c],
                device_id=x_peer, device_id_type=pl.DeviceIdType.MESH,
            )

        def y_rdma(c):
            return pltpu.make_async_remote_copy(
                src_ref=y_send.at[c], dst_ref=y_recv.at[c],
                send_sem=y_send_sems.at[c], recv_sem=y_recv_sems.at[c],
                device_id=y_peer, device_id_type=pl.DeviceIdType.MESH,
            )

        def out_copy(c, half_owner, sem):
            rows = pl.ds(half_owner * HALF + c * CH, CH)
            return pltpu.make_async_copy(
                out_stage.at[sem], out_ref.at[rows, :], copy_sems.at[sem],
            )

        b_bf16 = b_ref[...].astype(jnp.bfloat16)

        partials = []
        for c in range(NC):
            rows = pl.ds(my_y * HALF + c * CH, CH)
            p = jnp.dot(
                a_ref[rows, :].astype(jnp.bfloat16), b_bf16,
                preferred_element_type=jnp.float32,
            )
            partials.append(p)
            x_send[c] = _quant(p, S_X)
            if c == 0:
                pl.semaphore_wait(barrier_sem, 2)
            x_rdma(c).start()

        for c in range(NC):
            x_rdma(c).wait_recv()
            red = partials[c] + x_recv[c].astype(jnp.float32) * S_X
            y_send[c] = _quant(red, S_Y)
            y_rdma(c).start()
            out_stage[c] = red.astype(jnp.bfloat16)
            out_copy(c, my_y, c).start()

        for c in range(NC):
            y_rdma(c).wait_recv()
            out_stage[NC + c] = (
                y_recv[c].astype(jnp.float32) * S_Y
            ).astype(jnp.bfloat16)
            out_copy(c, 1 - my_y, NC + c).start()

        for c in range(NC):
            out_copy(c, my_y, c).wait()
        for c in range(NC):
            out_copy(c, 1 - my_y, NC + c).wait()
        for c in range(NC):
            x_rdma(c).wait_send()
            y_rdma(c).wait_send()

    return pl.pallas_call(
        body,
        out_shape=jax.ShapeDtypeStruct((M, N), jnp.bfloat16),
        in_specs=[
            pl.BlockSpec(memory_space=pltpu.VMEM),
            pl.BlockSpec(memory_space=pltpu.VMEM),
        ],
        out_specs=pl.BlockSpec(memory_space=pltpu.MemorySpace.HBM),
        scratch_shapes=[
            pltpu.VMEM((NC, CH, N), jnp.int8),
            pltpu.VMEM((NC, CH, N), jnp.int8),
            pltpu.VMEM((NC, CH, N), jnp.int8),
            pltpu.VMEM((NC, CH, N), jnp.int8),
            pltpu.VMEM((2 * NC, CH, N), jnp.bfloat16),
            pltpu.SemaphoreType.DMA((NC,)),
            pltpu.SemaphoreType.DMA((NC,)),
            pltpu.SemaphoreType.DMA((NC,)),
            pltpu.SemaphoreType.DMA((NC,)),
            pltpu.SemaphoreType.DMA((2 * NC,)),
        ],
        compiler_params=pltpu.CompilerParams(collective_id=0),
    )(A, B)
